# speedup vs baseline: 2.8766x; 2.8766x over previous
import functools

import jax
import jax.numpy as jnp
from jax import lax
from jax.experimental import pallas as pl
from jax.experimental.pallas import tpu as pltpu
from jax.experimental.pallas import tpu_sc as plsc


@functools.lru_cache(maxsize=None)
def _make_sc_gather(N, E, D, dtype_name):
    dtype = jnp.dtype(dtype_name)
    info = plsc.get_sparse_core_info()
    NC, NS = info.num_cores, info.num_subcores
    NW = NC * NS
    assert E % NW == 0
    epw = E // NW
    C = 80
    assert epw % C == 0 and C % 8 == 0
    n_chunks = epw // C

    mesh = plsc.VectorSubcoreMesh(core_axis_name="c", subcore_axis_name="s")

    @functools.partial(
        pl.kernel,
        out_type=(
            jax.ShapeDtypeStruct((E, D), dtype),
            jax.ShapeDtypeStruct((E, D), dtype),
        ),
        mesh=mesh,
        scratch_types=[
            pltpu.VMEM((C,), jnp.int32),
            pltpu.VMEM((C,), jnp.int32),
            pltpu.VMEM((C, D), dtype),
            pltpu.VMEM((C, D), dtype),
            pltpu.SemaphoreType.DMA,
            pltpu.SemaphoreType.DMA,
        ],
    )
    def sc_gather(x_hbm, src_hbm, dst_hbm, hs_hbm, hd_hbm,
                  idx_s, idx_d, rows_s, rows_d, sem_s, sem_d):
        wid = lax.axis_index("c") * NS + lax.axis_index("s")
        base_w = wid * epw

        def chunk(ci, carry):
            base = base_w + ci * C
            pltpu.sync_copy(src_hbm.at[pl.ds(base, C)], idx_s)
            pltpu.sync_copy(dst_hbm.at[pl.ds(base, C)], idx_d)
            cs = pltpu.async_copy(x_hbm.at[idx_s], rows_s, sem_s)
            cd = pltpu.async_copy(x_hbm.at[idx_d], rows_d, sem_d)
            cs.wait()
            pltpu.sync_copy(rows_s, hs_hbm.at[pl.ds(base, C)])
            cd.wait()
            pltpu.sync_copy(rows_d, hd_hbm.at[pl.ds(base, C)])
            return carry

        lax.fori_loop(0, n_chunks, chunk, 0)

    return sc_gather


def _tc_body(hs_ref, hd_ref, e_ref, g_ref, b_ref, w1_ref, b1_ref,
             w2_ref, b2_ref, out_ref, *, IN):
    hs = hs_ref[...]
    hd = hd_ref[...]
    e = e_ref[...]
    pair = jnp.concatenate([hs + hd, hs * hd, e], axis=1)
    inv = 1.0 / IN
    mu = jnp.sum(pair, axis=1, keepdims=True) * inv
    cen = pair - mu
    var = jnp.sum(cen * cen, axis=1, keepdims=True) * inv
    pairn = cen * lax.rsqrt(var + 1e-5) * g_ref[...] + b_ref[...]
    h = jnp.dot(pairn, w1_ref[...], preferred_element_type=jnp.float32)
    h = h + b1_ref[...]
    h = 0.5 * h * (1.0 + lax.erf(h * 0.7071067811865476))
    delta = jnp.dot(h, w2_ref[...], preferred_element_type=jnp.float32)
    out_ref[...] = e + delta + b2_ref[...]


@functools.lru_cache(maxsize=None)
def _make_tc_mlp(E, D, IN, HID, dtype_name, interpret=False):
    dtype = jnp.dtype(dtype_name)
    BE = 640
    assert E % BE == 0
    grid = (E // BE,)

    def edge_blk(i):
        return (i, 0)

    def fixed(i):
        return (0, 0)

    return pl.pallas_call(
        functools.partial(_tc_body, IN=IN),
        grid=grid,
        in_specs=[
            pl.BlockSpec((BE, D), edge_blk),
            pl.BlockSpec((BE, D), edge_blk),
            pl.BlockSpec((BE, D), edge_blk),
            pl.BlockSpec((1, IN), fixed),
            pl.BlockSpec((1, IN), fixed),
            pl.BlockSpec((IN, HID), fixed),
            pl.BlockSpec((1, HID), fixed),
            pl.BlockSpec((HID, D), fixed),
            pl.BlockSpec((1, D), fixed),
        ],
        out_specs=pl.BlockSpec((BE, D), edge_blk),
        out_shape=jax.ShapeDtypeStruct((E, D), dtype),
        interpret=interpret,
    )


def kernel(x, edge_index, edge_attr, ln_gamma, ln_beta, W1, b1, W2, b2):
    N, D = x.shape
    E = edge_attr.shape[0]
    IN, HID = W1.shape
    src = edge_index[0]
    dst = edge_index[1]
    sc_gather = _make_sc_gather(N, E, D, x.dtype.name)
    h_src, h_dst = sc_gather(x, src, dst)
    tc_mlp = _make_tc_mlp(E, D, IN, HID, x.dtype.name)
    return tc_mlp(h_src, h_dst, edge_attr,
                  ln_gamma.reshape(1, IN), ln_beta.reshape(1, IN),
                  W1, b1.reshape(1, HID), W2, b2.reshape(1, D))


# speedup vs baseline: 3.4331x; 1.1935x over previous
import functools

import jax
import jax.numpy as jnp
from jax import lax
from jax.experimental import pallas as pl
from jax.experimental.pallas import tpu as pltpu
from jax.experimental.pallas import tpu_sc as plsc


@functools.lru_cache(maxsize=None)
def _make_sc_gather(N, E, D, dtype_name):
    dtype = jnp.dtype(dtype_name)
    info = plsc.get_sparse_core_info()
    NC, NS = info.num_cores, info.num_subcores
    NW = NC * NS
    assert E % NW == 0
    epw = E // NW
    C = 80
    assert epw % C == 0 and C % 8 == 0
    n_chunks = epw // C

    mesh = plsc.VectorSubcoreMesh(core_axis_name="c", subcore_axis_name="s")

    @functools.partial(
        pl.kernel,
        out_type=(
            jax.ShapeDtypeStruct((E, D), dtype),
            jax.ShapeDtypeStruct((E, D), dtype),
        ),
        mesh=mesh,
        scratch_types=[
            pltpu.VMEM((C,), jnp.int32),
            pltpu.VMEM((C,), jnp.int32),
            pltpu.VMEM((C, D), dtype),
            pltpu.VMEM((C, D), dtype),
            pltpu.SemaphoreType.DMA,
            pltpu.SemaphoreType.DMA,
        ],
    )
    def sc_gather(x_hbm, src_hbm, dst_hbm, hs_hbm, hd_hbm,
                  idx_s, idx_d, rows_s, rows_d, sem_s, sem_d):
        wid = lax.axis_index("c") * NS + lax.axis_index("s")
        base_w = wid * epw

        def chunk(ci, carry):
            base = base_w + ci * C
            pltpu.sync_copy(src_hbm.at[pl.ds(base, C)], idx_s)
            pltpu.sync_copy(dst_hbm.at[pl.ds(base, C)], idx_d)
            cs = pltpu.async_copy(x_hbm.at[idx_s], rows_s, sem_s)
            cd = pltpu.async_copy(x_hbm.at[idx_d], rows_d, sem_d)
            cs.wait()
            pltpu.sync_copy(rows_s, hs_hbm.at[pl.ds(base, C)])
            cd.wait()
            pltpu.sync_copy(rows_d, hd_hbm.at[pl.ds(base, C)])
            return carry

        lax.fori_loop(0, n_chunks, chunk, 0)

    return sc_gather


def _tc_body(hs_ref, hd_ref, e_ref, g_ref, b_ref, w1_ref, b1_ref,
             w2_ref, b2_ref, out_ref, *, IN):
    hs = hs_ref[...]
    hd = hd_ref[...]
    e = e_ref[...]
    pair = jnp.concatenate([hs + hd, hs * hd, e], axis=1)
    inv = 1.0 / IN
    mu = jnp.sum(pair, axis=1, keepdims=True) * inv
    cen = pair - mu
    var = jnp.sum(cen * cen, axis=1, keepdims=True) * inv
    pairn = cen * lax.rsqrt(var + 1e-5) * g_ref[...] + b_ref[...]
    h = jnp.dot(pairn, w1_ref[...], preferred_element_type=jnp.float32)
    h = h + b1_ref[...]
    h = 0.5 * h * (1.0 + lax.erf(h * 0.7071067811865476))
    delta = jnp.dot(h, w2_ref[...], preferred_element_type=jnp.float32)
    out_ref[...] = e + delta + b2_ref[...]


@functools.lru_cache(maxsize=None)
def _make_tc_mlp(E, D, IN, HID, dtype_name, interpret=False):
    dtype = jnp.dtype(dtype_name)
    BE = 1280
    assert E % BE == 0
    grid = (E // BE,)

    def edge_blk(i):
        return (i, 0)

    def fixed(i):
        return (0, 0)

    return pl.pallas_call(
        functools.partial(_tc_body, IN=IN),
        grid=grid,
        in_specs=[
            pl.BlockSpec((BE, D), edge_blk),
            pl.BlockSpec((BE, D), edge_blk),
            pl.BlockSpec((BE, D), edge_blk),
            pl.BlockSpec((1, IN), fixed),
            pl.BlockSpec((1, IN), fixed),
            pl.BlockSpec((IN, HID), fixed),
            pl.BlockSpec((1, HID), fixed),
            pl.BlockSpec((HID, D), fixed),
            pl.BlockSpec((1, D), fixed),
        ],
        out_specs=pl.BlockSpec((BE, D), edge_blk),
        out_shape=jax.ShapeDtypeStruct((E, D), dtype),
        interpret=interpret,
    )


def kernel(x, edge_index, edge_attr, ln_gamma, ln_beta, W1, b1, W2, b2):
    N, D = x.shape
    E = edge_attr.shape[0]
    IN, HID = W1.shape
    src = edge_index[0]
    dst = edge_index[1]
    sc_gather = _make_sc_gather(N, E, D, x.dtype.name)
    h_src, h_dst = sc_gather(x, src, dst)
    tc_mlp = _make_tc_mlp(E, D, IN, HID, x.dtype.name)
    return tc_mlp(h_src, h_dst, edge_attr,
                  ln_gamma.reshape(1, IN), ln_beta.reshape(1, IN),
                  W1, b1.reshape(1, HID), W2, b2.reshape(1, D))


# speedup vs baseline: 3.8277x; 1.1149x over previous
import functools

import jax
import jax.numpy as jnp
from jax import lax
from jax.experimental import pallas as pl
from jax.experimental.pallas import tpu as pltpu
from jax.experimental.pallas import tpu_sc as plsc


@functools.lru_cache(maxsize=None)
def _make_sc_gather(N, E, D, dtype_name):
    dtype = jnp.dtype(dtype_name)
    info = plsc.get_sparse_core_info()
    NC, NS = info.num_cores, info.num_subcores
    NW = NC * NS
    assert E % NW == 0
    epw = E // NW
    C = 80
    assert epw % C == 0 and C % 8 == 0
    n_chunks = epw // C

    mesh = plsc.VectorSubcoreMesh(core_axis_name="c", subcore_axis_name="s")

    @functools.partial(
        pl.kernel,
        out_type=(
            jax.ShapeDtypeStruct((E, D), dtype),
            jax.ShapeDtypeStruct((E, D), dtype),
        ),
        mesh=mesh,
        scratch_types=[
            pltpu.VMEM((C,), jnp.int32),
            pltpu.VMEM((C,), jnp.int32),
            pltpu.VMEM((C, D), dtype),
            pltpu.VMEM((C, D), dtype),
            pltpu.SemaphoreType.DMA,
            pltpu.SemaphoreType.DMA,
        ],
    )
    def sc_gather(x_hbm, src_hbm, dst_hbm, hs_hbm, hd_hbm,
                  idx_s, idx_d, rows_s, rows_d, sem_s, sem_d):
        wid = lax.axis_index("c") * NS + lax.axis_index("s")
        base_w = wid * epw

        def chunk(ci, carry):
            base = base_w + ci * C
            pltpu.sync_copy(src_hbm.at[pl.ds(base, C)], idx_s)
            pltpu.sync_copy(dst_hbm.at[pl.ds(base, C)], idx_d)
            cs = pltpu.async_copy(x_hbm.at[idx_s], rows_s, sem_s)
            cd = pltpu.async_copy(x_hbm.at[idx_d], rows_d, sem_d)
            cs.wait()
            pltpu.sync_copy(rows_s, hs_hbm.at[pl.ds(base, C)])
            cd.wait()
            pltpu.sync_copy(rows_d, hd_hbm.at[pl.ds(base, C)])
            return carry

        lax.fori_loop(0, n_chunks, chunk, 0)

    return sc_gather


def _tc_body(hs_ref, hd_ref, e_ref, g_ref, b_ref, w1_ref, b1_ref,
             w2_ref, b2_ref, out_ref, *, IN):
    hs = hs_ref[...]
    hd = hd_ref[...]
    e = e_ref[...]
    pair = jnp.concatenate([hs + hd, hs * hd, e], axis=1)
    inv = 1.0 / IN
    mu = jnp.sum(pair, axis=1, keepdims=True) * inv
    cen = pair - mu
    var = jnp.sum(cen * cen, axis=1, keepdims=True) * inv
    pairn = cen * lax.rsqrt(var + 1e-5) * g_ref[...] + b_ref[...]
    h = jnp.dot(pairn, w1_ref[...], preferred_element_type=jnp.float32)
    h = h + b1_ref[...]
    h = 0.5 * h * (1.0 + lax.erf(h * 0.7071067811865476))
    delta = jnp.dot(h, w2_ref[...], preferred_element_type=jnp.float32)
    out_ref[...] = e + delta + b2_ref[...]


@functools.lru_cache(maxsize=None)
def _make_tc_mlp(E, D, IN, HID, dtype_name, interpret=False):
    dtype = jnp.dtype(dtype_name)
    BE = 2560
    assert E % BE == 0
    grid = (E // BE,)

    def edge_blk(i):
        return (i, 0)

    def fixed(i):
        return (0, 0)

    return pl.pallas_call(
        functools.partial(_tc_body, IN=IN),
        grid=grid,
        in_specs=[
            pl.BlockSpec((BE, D), edge_blk),
            pl.BlockSpec((BE, D), edge_blk),
            pl.BlockSpec((BE, D), edge_blk),
            pl.BlockSpec((1, IN), fixed),
            pl.BlockSpec((1, IN), fixed),
            pl.BlockSpec((IN, HID), fixed),
            pl.BlockSpec((1, HID), fixed),
            pl.BlockSpec((HID, D), fixed),
            pl.BlockSpec((1, D), fixed),
        ],
        out_specs=pl.BlockSpec((BE, D), edge_blk),
        out_shape=jax.ShapeDtypeStruct((E, D), dtype),
        interpret=interpret,
    )


def kernel(x, edge_index, edge_attr, ln_gamma, ln_beta, W1, b1, W2, b2):
    N, D = x.shape
    E = edge_attr.shape[0]
    IN, HID = W1.shape
    src = edge_index[0]
    dst = edge_index[1]
    sc_gather = _make_sc_gather(N, E, D, x.dtype.name)
    h_src, h_dst = sc_gather(x, src, dst)
    tc_mlp = _make_tc_mlp(E, D, IN, HID, x.dtype.name)
    return tc_mlp(h_src, h_dst, edge_attr,
                  ln_gamma.reshape(1, IN), ln_beta.reshape(1, IN),
                  W1, b1.reshape(1, HID), W2, b2.reshape(1, D))


# speedup vs baseline: 4.0856x; 1.0674x over previous
import functools

import jax
import jax.numpy as jnp
from jax import lax
from jax.experimental import pallas as pl
from jax.experimental.pallas import tpu as pltpu
from jax.experimental.pallas import tpu_sc as plsc


@functools.lru_cache(maxsize=None)
def _make_sc_gather(N, E, D, dtype_name):
    dtype = jnp.dtype(dtype_name)
    info = plsc.get_sparse_core_info()
    NC, NS = info.num_cores, info.num_subcores
    NW = NC * NS
    assert E % NW == 0
    epw = E // NW
    C = 80
    assert epw % C == 0 and C % 8 == 0
    n_chunks = epw // C

    mesh = plsc.VectorSubcoreMesh(core_axis_name="c", subcore_axis_name="s")

    @functools.partial(
        pl.kernel,
        out_type=(
            jax.ShapeDtypeStruct((E, D), dtype),
            jax.ShapeDtypeStruct((E, D), dtype),
        ),
        mesh=mesh,
        scratch_types=[
            pltpu.VMEM((C,), jnp.int32),
            pltpu.VMEM((C,), jnp.int32),
            pltpu.VMEM((C, D), dtype),
            pltpu.VMEM((C, D), dtype),
            pltpu.SemaphoreType.DMA,
            pltpu.SemaphoreType.DMA,
        ],
    )
    def sc_gather(x_hbm, src_hbm, dst_hbm, hs_hbm, hd_hbm,
                  idx_s, idx_d, rows_s, rows_d, sem_s, sem_d):
        wid = lax.axis_index("c") * NS + lax.axis_index("s")
        base_w = wid * epw

        def chunk(ci, carry):
            base = base_w + ci * C
            pltpu.sync_copy(src_hbm.at[pl.ds(base, C)], idx_s)
            pltpu.sync_copy(dst_hbm.at[pl.ds(base, C)], idx_d)
            cs = pltpu.async_copy(x_hbm.at[idx_s], rows_s, sem_s)
            cd = pltpu.async_copy(x_hbm.at[idx_d], rows_d, sem_d)
            cs.wait()
            pltpu.sync_copy(rows_s, hs_hbm.at[pl.ds(base, C)])
            cd.wait()
            pltpu.sync_copy(rows_d, hd_hbm.at[pl.ds(base, C)])
            return carry

        lax.fori_loop(0, n_chunks, chunk, 0)

    return sc_gather


def _tc_body(hs_ref, hd_ref, e_ref, g_ref, b_ref, w1_ref, b1_ref,
             w2_ref, b2_ref, out_ref, *, IN):
    hs = hs_ref[...]
    hd = hd_ref[...]
    e = e_ref[...]
    pair = jnp.concatenate([hs + hd, hs * hd, e], axis=1)
    inv = 1.0 / IN
    mu = jnp.sum(pair, axis=1, keepdims=True) * inv
    cen = pair - mu
    var = jnp.sum(cen * cen, axis=1, keepdims=True) * inv
    pairn = cen * lax.rsqrt(var + 1e-5) * g_ref[...] + b_ref[...]
    h = jnp.dot(pairn, w1_ref[...], preferred_element_type=jnp.float32)
    h = h + b1_ref[...]
    h = 0.5 * h * (1.0 + lax.erf(h * 0.7071067811865476))
    delta = jnp.dot(h, w2_ref[...], preferred_element_type=jnp.float32)
    out_ref[...] = e + delta + b2_ref[...]


@functools.lru_cache(maxsize=None)
def _make_tc_mlp(E, D, IN, HID, dtype_name, interpret=False):
    dtype = jnp.dtype(dtype_name)
    BE = 6400
    assert E % BE == 0
    grid = (E // BE,)

    def edge_blk(i):
        return (i, 0)

    def fixed(i):
        return (0, 0)

    return pl.pallas_call(
        functools.partial(_tc_body, IN=IN),
        grid=grid,
        in_specs=[
            pl.BlockSpec((BE, D), edge_blk),
            pl.BlockSpec((BE, D), edge_blk),
            pl.BlockSpec((BE, D), edge_blk),
            pl.BlockSpec((1, IN), fixed),
            pl.BlockSpec((1, IN), fixed),
            pl.BlockSpec((IN, HID), fixed),
            pl.BlockSpec((1, HID), fixed),
            pl.BlockSpec((HID, D), fixed),
            pl.BlockSpec((1, D), fixed),
        ],
        out_specs=pl.BlockSpec((BE, D), edge_blk),
        out_shape=jax.ShapeDtypeStruct((E, D), dtype),
        interpret=interpret,
    )


def kernel(x, edge_index, edge_attr, ln_gamma, ln_beta, W1, b1, W2, b2):
    N, D = x.shape
    E = edge_attr.shape[0]
    IN, HID = W1.shape
    src = edge_index[0]
    dst = edge_index[1]
    sc_gather = _make_sc_gather(N, E, D, x.dtype.name)
    h_src, h_dst = sc_gather(x, src, dst)
    tc_mlp = _make_tc_mlp(E, D, IN, HID, x.dtype.name)
    return tc_mlp(h_src, h_dst, edge_attr,
                  ln_gamma.reshape(1, IN), ln_beta.reshape(1, IN),
                  W1, b1.reshape(1, HID), W2, b2.reshape(1, D))


# speedup vs baseline: 4.7401x; 1.1602x over previous
import functools

import jax
import jax.numpy as jnp
from jax import lax
from jax.experimental import pallas as pl
from jax.experimental.pallas import tpu as pltpu
from jax.experimental.pallas import tpu_sc as plsc


@functools.lru_cache(maxsize=None)
def _make_sc_gather(N, Ec, D, dtype_name):
    dtype = jnp.dtype(dtype_name)
    info = plsc.get_sparse_core_info()
    NC, NS = info.num_cores, info.num_subcores
    NW = NC * NS
    assert Ec % NW == 0
    epw = Ec // NW
    C = next(c for c in range(128, 7, -8) if epw % c == 0)
    n_chunks = epw // C

    mesh = plsc.VectorSubcoreMesh(core_axis_name="c", subcore_axis_name="s")

    @functools.partial(
        pl.kernel,
        out_type=(
            jax.ShapeDtypeStruct((Ec, D), dtype),
            jax.ShapeDtypeStruct((Ec, D), dtype),
        ),
        mesh=mesh,
        scratch_types=[
            pltpu.VMEM((C,), jnp.int32),
            pltpu.VMEM((C,), jnp.int32),
            pltpu.VMEM((C, D), dtype),
            pltpu.VMEM((C, D), dtype),
            pltpu.SemaphoreType.DMA,
            pltpu.SemaphoreType.DMA,
        ],
    )
    def sc_gather(x_hbm, src_hbm, dst_hbm, hs_hbm, hd_hbm,
                  idx_s, idx_d, rows_s, rows_d, sem_s, sem_d):
        wid = lax.axis_index("c") * NS + lax.axis_index("s")
        base_w = wid * epw

        def chunk(ci, carry):
            base = base_w + ci * C
            pltpu.sync_copy(src_hbm.at[pl.ds(base, C)], idx_s)
            pltpu.sync_copy(dst_hbm.at[pl.ds(base, C)], idx_d)
            cs = pltpu.async_copy(x_hbm.at[idx_s], rows_s, sem_s)
            cd = pltpu.async_copy(x_hbm.at[idx_d], rows_d, sem_d)
            cs.wait()
            pltpu.sync_copy(rows_s, hs_hbm.at[pl.ds(base, C)])
            cd.wait()
            pltpu.sync_copy(rows_d, hd_hbm.at[pl.ds(base, C)])
            return carry

        lax.fori_loop(0, n_chunks, chunk, 0)

    return sc_gather


def _tc_body(hs_ref, hd_ref, e_ref, g_ref, b_ref, w1_ref, b1_ref,
             w2_ref, b2_ref, out_ref, *, IN):
    hs = hs_ref[...]
    hd = hd_ref[...]
    e = e_ref[...]
    pair = jnp.concatenate([hs + hd, hs * hd, e], axis=1)
    inv = 1.0 / IN
    mu = jnp.sum(pair, axis=1, keepdims=True) * inv
    cen = pair - mu
    var = jnp.sum(cen * cen, axis=1, keepdims=True) * inv
    pairn = cen * lax.rsqrt(var + 1e-5) * g_ref[...] + b_ref[...]
    h = jnp.dot(pairn.astype(jnp.bfloat16), w1_ref[...],
                preferred_element_type=jnp.float32)
    h = h + b1_ref[...]
    h = 0.5 * h * (1.0 + lax.erf(h * 0.7071067811865476))
    delta = jnp.dot(h.astype(jnp.bfloat16), w2_ref[...],
                    preferred_element_type=jnp.float32)
    out_ref[...] = e + delta + b2_ref[...]


def _tc_body_carry(carry_ref, *rest, IN):
    _tc_body(*rest, IN=IN)


@functools.lru_cache(maxsize=None)
def _make_tc_chunk(E, Ec, D, IN, HID, k, with_carry, dtype_name,
                   interpret=False):
    dtype = jnp.dtype(dtype_name)
    BE = 6400
    assert Ec % BE == 0
    nb = Ec // BE
    off = k * nb

    def chunk_blk(i):
        return (i, 0)

    def full_blk(i):
        return (i + off, 0)

    def fixed(i):
        return (0, 0)

    in_specs = [
        pl.BlockSpec((BE, D), chunk_blk),
        pl.BlockSpec((BE, D), chunk_blk),
        pl.BlockSpec((BE, D), full_blk),
        pl.BlockSpec((1, IN), fixed),
        pl.BlockSpec((1, IN), fixed),
        pl.BlockSpec((IN, HID), fixed),
        pl.BlockSpec((1, HID), fixed),
        pl.BlockSpec((HID, D), fixed),
        pl.BlockSpec((1, D), fixed),
    ]
    body = functools.partial(_tc_body, IN=IN)
    aliases = {}
    if with_carry:
        in_specs = [pl.BlockSpec(memory_space=pl.ANY)] + in_specs
        body = functools.partial(_tc_body_carry, IN=IN)
        aliases = {0: 0}

    return pl.pallas_call(
        body,
        grid=(nb,),
        in_specs=in_specs,
        out_specs=pl.BlockSpec((BE, D), full_blk),
        out_shape=jax.ShapeDtypeStruct((E, D), dtype),
        input_output_aliases=aliases,
        interpret=interpret,
    )


def kernel(x, edge_index, edge_attr, ln_gamma, ln_beta, W1, b1, W2, b2):
    N, D = x.shape
    E = edge_attr.shape[0]
    IN, HID = W1.shape
    src = edge_index[0]
    dst = edge_index[1]
    K = 5
    Ec = E // K
    assert E % K == 0
    sc_gather = _make_sc_gather(N, Ec, D, x.dtype.name)
    g2 = ln_gamma.reshape(1, IN)
    bt = ln_beta.reshape(1, IN)
    w1b = W1.astype(jnp.bfloat16)
    b1r = b1.reshape(1, HID)
    w2b = W2.astype(jnp.bfloat16)
    b2r = b2.reshape(1, D)
    out = None
    for k in range(K):
        s_k = lax.slice_in_dim(src, k * Ec, (k + 1) * Ec)
        d_k = lax.slice_in_dim(dst, k * Ec, (k + 1) * Ec)
        hs_k, hd_k = sc_gather(x, s_k, d_k)
        tc = _make_tc_chunk(E, Ec, D, IN, HID, k, k > 0, x.dtype.name)
        args = (hs_k, hd_k, edge_attr, g2, bt, w1b, b1r, w2b, b2r)
        out = tc(*args) if k == 0 else tc(out, *args)
    return out


# speedup vs baseline: 5.4524x; 1.1503x over previous
import functools

import jax
import jax.numpy as jnp
from jax import lax
from jax.experimental import pallas as pl
from jax.experimental.pallas import tpu as pltpu
from jax.experimental.pallas import tpu_sc as plsc


@functools.lru_cache(maxsize=None)
def _make_sc_gather(N, Ec, D, dtype_name):
    dtype = jnp.dtype(dtype_name)
    info = plsc.get_sparse_core_info()
    NC, NS = info.num_cores, info.num_subcores
    NW = NC * NS
    assert Ec % NW == 0
    epw = Ec // NW
    C = next(c for c in range(128, 7, -8) if epw % c == 0)
    n_chunks = epw // C

    mesh = plsc.VectorSubcoreMesh(core_axis_name="c", subcore_axis_name="s")

    @functools.partial(
        pl.kernel,
        out_type=(
            jax.ShapeDtypeStruct((Ec, D), dtype),
            jax.ShapeDtypeStruct((Ec, D), dtype),
        ),
        mesh=mesh,
        scratch_types=[
            pltpu.VMEM((C,), jnp.int32),
            pltpu.VMEM((C,), jnp.int32),
            pltpu.VMEM((C, D), dtype),
            pltpu.VMEM((C, D), dtype),
            pltpu.VMEM_SHARED((N, D), dtype),
            pltpu.SemaphoreType.DMA,
            pltpu.SemaphoreType.DMA,
        ],
    )
    def sc_gather(x_hbm, src_hbm, dst_hbm, hs_hbm, hd_hbm,
                  idx_s, idx_d, rows_s, rows_d, x_sp, sem_s, sem_d):
        wid = lax.axis_index("c") * NS + lax.axis_index("s")
        base_w = wid * epw

        @pl.when(lax.axis_index("s") == 0)
        def _stage():
            pltpu.sync_copy(x_hbm, x_sp)

        plsc.subcore_barrier()

        def chunk(ci, carry):
            base = base_w + ci * C
            pltpu.sync_copy(src_hbm.at[pl.ds(base, C)], idx_s)
            pltpu.sync_copy(dst_hbm.at[pl.ds(base, C)], idx_d)
            cs = pltpu.async_copy(x_sp.at[idx_s], rows_s, sem_s)
            cd = pltpu.async_copy(x_sp.at[idx_d], rows_d, sem_d)
            cs.wait()
            pltpu.sync_copy(rows_s, hs_hbm.at[pl.ds(base, C)])
            cd.wait()
            pltpu.sync_copy(rows_d, hd_hbm.at[pl.ds(base, C)])
            return carry

        lax.fori_loop(0, n_chunks, chunk, 0)

    return sc_gather


def _tc_body(hs_ref, hd_ref, e_ref, g_ref, b_ref, w1_ref, b1_ref,
             w2_ref, b2_ref, out_ref, *, IN):
    hs = hs_ref[...]
    hd = hd_ref[...]
    e = e_ref[...]
    pair = jnp.concatenate([hs + hd, hs * hd, e], axis=1)
    inv = 1.0 / IN
    mu = jnp.sum(pair, axis=1, keepdims=True) * inv
    cen = pair - mu
    var = jnp.sum(cen * cen, axis=1, keepdims=True) * inv
    pairn = cen * lax.rsqrt(var + 1e-5) * g_ref[...] + b_ref[...]
    h = jnp.dot(pairn.astype(jnp.bfloat16), w1_ref[...],
                preferred_element_type=jnp.float32)
    h = h + b1_ref[...]
    h = 0.5 * h * (1.0 + lax.erf(h * 0.7071067811865476))
    delta = jnp.dot(h.astype(jnp.bfloat16), w2_ref[...],
                    preferred_element_type=jnp.float32)
    out_ref[...] = e + delta + b2_ref[...]


def _tc_body_carry(carry_ref, *rest, IN):
    _tc_body(*rest, IN=IN)


@functools.lru_cache(maxsize=None)
def _make_tc_chunk(E, Ec, D, IN, HID, k, with_carry, dtype_name,
                   interpret=False):
    dtype = jnp.dtype(dtype_name)
    BE = 6400
    assert Ec % BE == 0
    nb = Ec // BE
    off = k * nb

    def chunk_blk(i):
        return (i, 0)

    def full_blk(i):
        return (i + off, 0)

    def fixed(i):
        return (0, 0)

    in_specs = [
        pl.BlockSpec((BE, D), chunk_blk),
        pl.BlockSpec((BE, D), chunk_blk),
        pl.BlockSpec((BE, D), full_blk),
        pl.BlockSpec((1, IN), fixed),
        pl.BlockSpec((1, IN), fixed),
        pl.BlockSpec((IN, HID), fixed),
        pl.BlockSpec((1, HID), fixed),
        pl.BlockSpec((HID, D), fixed),
        pl.BlockSpec((1, D), fixed),
    ]
    body = functools.partial(_tc_body, IN=IN)
    aliases = {}
    if with_carry:
        in_specs = [pl.BlockSpec(memory_space=pl.ANY)] + in_specs
        body = functools.partial(_tc_body_carry, IN=IN)
        aliases = {0: 0}

    return pl.pallas_call(
        body,
        grid=(nb,),
        in_specs=in_specs,
        out_specs=pl.BlockSpec((BE, D), full_blk),
        out_shape=jax.ShapeDtypeStruct((E, D), dtype),
        input_output_aliases=aliases,
        interpret=interpret,
    )


def kernel(x, edge_index, edge_attr, ln_gamma, ln_beta, W1, b1, W2, b2):
    N, D = x.shape
    E = edge_attr.shape[0]
    IN, HID = W1.shape
    src = edge_index[0]
    dst = edge_index[1]
    K = 5
    Ec = E // K
    assert E % K == 0
    sc_gather = _make_sc_gather(N, Ec, D, x.dtype.name)
    g2 = ln_gamma.reshape(1, IN)
    bt = ln_beta.reshape(1, IN)
    w1b = W1.astype(jnp.bfloat16)
    b1r = b1.reshape(1, HID)
    w2b = W2.astype(jnp.bfloat16)
    b2r = b2.reshape(1, D)
    out = None
    for k in range(K):
        s_k = lax.slice_in_dim(src, k * Ec, (k + 1) * Ec)
        d_k = lax.slice_in_dim(dst, k * Ec, (k + 1) * Ec)
        hs_k, hd_k = sc_gather(x, s_k, d_k)
        tc = _make_tc_chunk(E, Ec, D, IN, HID, k, k > 0, x.dtype.name)
        args = (hs_k, hd_k, edge_attr, g2, bt, w1b, b1r, w2b, b2r)
        out = tc(*args) if k == 0 else tc(out, *args)
    return out


# speedup vs baseline: 6.9243x; 1.2700x over previous
import functools

import jax
import jax.numpy as jnp
from jax import lax
from jax.experimental import pallas as pl
from jax.experimental.pallas import tpu as pltpu
from jax.experimental.pallas import tpu_sc as plsc


@functools.lru_cache(maxsize=None)
def _make_sc_gather(N, Ec, D, dtype_name):
    dtype = jnp.dtype(dtype_name)
    info = plsc.get_sparse_core_info()
    NC, NS = info.num_cores, info.num_subcores
    NW = NC * NS
    assert Ec % NW == 0
    epw = Ec // NW
    C = next(c for c in range(128, 7, -8) if epw % c == 0)
    n = epw // C
    assert n >= 3 and n % 2 == 1
    SROW = next(s for s in (16, 10, 8, 5, 4, 2, 1)
                if N % s == 0 and (N // s) % 8 == 0)
    rpt = N // SROW

    mesh = plsc.VectorSubcoreMesh(core_axis_name="c", subcore_axis_name="s")

    @functools.partial(
        pl.kernel,
        out_type=(
            jax.ShapeDtypeStruct((Ec, D), dtype),
            jax.ShapeDtypeStruct((Ec, D), dtype),
        ),
        mesh=mesh,
        scratch_types=[
            pltpu.VMEM((C,), jnp.int32),
            pltpu.VMEM((C,), jnp.int32),
            pltpu.VMEM((C,), jnp.int32),
            pltpu.VMEM((C,), jnp.int32),
            pltpu.VMEM((C, D), dtype),
            pltpu.VMEM((C, D), dtype),
            pltpu.VMEM((C, D), dtype),
            pltpu.VMEM((C, D), dtype),
            pltpu.VMEM_SHARED((N, D), dtype),
            pltpu.SemaphoreType.DMA,
            pltpu.SemaphoreType.DMA,
            pltpu.SemaphoreType.DMA,
            pltpu.SemaphoreType.DMA,
            pltpu.SemaphoreType.DMA,
            pltpu.SemaphoreType.DMA,
        ],
    )
    def sc_gather(x_hbm, src_hbm, dst_hbm, hs_hbm, hd_hbm,
                  i_s0, i_s1, i_d0, i_d1, r_s0, r_s1, r_d0, r_d1, x_sp,
                  sem_i0, sem_i1, sem_g0, sem_g1, sem_v0, sem_v1):
        idx_s = (i_s0, i_s1)
        idx_d = (i_d0, i_d1)
        rows_s = (r_s0, r_s1)
        rows_d = (r_d0, r_d1)
        sem_i = (sem_i0, sem_i1)
        sem_g = (sem_g0, sem_g1)
        sem_v = (sem_v0, sem_v1)
        cid = lax.axis_index("c")
        sid = lax.axis_index("s")
        wid = cid * NS + sid
        base_w = wid * epw

        @pl.when(sid < SROW)
        def _stage():
            pltpu.sync_copy(x_hbm.at[pl.ds(sid * rpt, rpt)],
                            x_sp.at[pl.ds(sid * rpt, rpt)])

        plsc.subcore_barrier()

        def issue_idx(c, p):
            base = base_w + c * C
            pltpu.async_copy(src_hbm.at[pl.ds(base, C)], idx_s[p], sem_i[p])
            pltpu.async_copy(dst_hbm.at[pl.ds(base, C)], idx_d[p], sem_i[p])

        def wait_idx(p):
            pltpu.make_async_copy(
                src_hbm.at[pl.ds(0, C)], idx_s[p], sem_i[p]).wait()
            pltpu.make_async_copy(
                dst_hbm.at[pl.ds(0, C)], idx_d[p], sem_i[p]).wait()

        def issue_gather(p):
            pltpu.async_copy(x_sp.at[idx_s[p]], rows_s[p], sem_g[p])
            pltpu.async_copy(x_sp.at[idx_d[p]], rows_d[p], sem_g[p])

        def wait_gather(p):
            pltpu.make_async_copy(
                x_sp.at[idx_s[p]], rows_s[p], sem_g[p]).wait()
            pltpu.make_async_copy(
                x_sp.at[idx_d[p]], rows_d[p], sem_g[p]).wait()

        def issue_scatter(c, p):
            base = base_w + c * C
            pltpu.async_copy(rows_s[p], hs_hbm.at[pl.ds(base, C)], sem_v[p])
            pltpu.async_copy(rows_d[p], hd_hbm.at[pl.ds(base, C)], sem_v[p])

        def wait_scatter(p):
            pltpu.make_async_copy(
                rows_s[p], hs_hbm.at[pl.ds(0, C)], sem_v[p]).wait()
            pltpu.make_async_copy(
                rows_d[p], hd_hbm.at[pl.ds(0, C)], sem_v[p]).wait()

        def tick(c, p):
            q = 1 - p
            wait_idx(q)

            @pl.when(c >= 1)
            def _(): wait_scatter(q)

            issue_gather(q)
            wait_gather(p)

            @pl.when(c + 2 < n)
            def _(): issue_idx(c + 2, p)

            issue_scatter(c, p)

        issue_idx(0, 0)
        issue_idx(1, 1)
        wait_idx(0)
        issue_gather(0)

        def pair(oi, carry):
            tick(2 * oi, 0)
            tick(2 * oi + 1, 1)
            return carry

        lax.fori_loop(0, (n - 1) // 2, pair, 0)

        wait_scatter(1)
        wait_gather(0)
        issue_scatter(n - 1, 0)
        wait_scatter(0)

    return sc_gather


def _tc_body(hs_ref, hd_ref, e_ref, g_ref, b_ref, w1_ref, b1_ref,
             w2_ref, b2_ref, out_ref, *, IN):
    hs = hs_ref[...]
    hd = hd_ref[...]
    e = e_ref[...]
    pair = jnp.concatenate([hs + hd, hs * hd, e], axis=1)
    inv = 1.0 / IN
    mu = jnp.sum(pair, axis=1, keepdims=True) * inv
    cen = pair - mu
    var = jnp.sum(cen * cen, axis=1, keepdims=True) * inv
    pairn = cen * lax.rsqrt(var + 1e-5) * g_ref[...] + b_ref[...]
    h = jnp.dot(pairn.astype(jnp.bfloat16), w1_ref[...],
                preferred_element_type=jnp.float32)
    h = h + b1_ref[...]
    h = 0.5 * h * (1.0 + lax.erf(h * 0.7071067811865476))
    delta = jnp.dot(h.astype(jnp.bfloat16), w2_ref[...],
                    preferred_element_type=jnp.float32)
    out_ref[...] = e + delta + b2_ref[...]


def _tc_body_carry(carry_ref, *rest, IN):
    _tc_body(*rest, IN=IN)


@functools.lru_cache(maxsize=None)
def _make_tc_chunk(E, Ec, D, IN, HID, k, with_carry, dtype_name,
                   interpret=False):
    dtype = jnp.dtype(dtype_name)
    BE = 6400
    assert Ec % BE == 0
    nb = Ec // BE
    off = k * nb

    def chunk_blk(i):
        return (i, 0)

    def full_blk(i):
        return (i + off, 0)

    def fixed(i):
        return (0, 0)

    in_specs = [
        pl.BlockSpec((BE, D), chunk_blk),
        pl.BlockSpec((BE, D), chunk_blk),
        pl.BlockSpec((BE, D), full_blk),
        pl.BlockSpec((1, IN), fixed),
        pl.BlockSpec((1, IN), fixed),
        pl.BlockSpec((IN, HID), fixed),
        pl.BlockSpec((1, HID), fixed),
        pl.BlockSpec((HID, D), fixed),
        pl.BlockSpec((1, D), fixed),
    ]
    body = functools.partial(_tc_body, IN=IN)
    aliases = {}
    if with_carry:
        in_specs = [pl.BlockSpec(memory_space=pl.ANY)] + in_specs
        body = functools.partial(_tc_body_carry, IN=IN)
        aliases = {0: 0}

    return pl.pallas_call(
        body,
        grid=(nb,),
        in_specs=in_specs,
        out_specs=pl.BlockSpec((BE, D), full_blk),
        out_shape=jax.ShapeDtypeStruct((E, D), dtype),
        input_output_aliases=aliases,
        interpret=interpret,
    )


def kernel(x, edge_index, edge_attr, ln_gamma, ln_beta, W1, b1, W2, b2):
    N, D = x.shape
    E = edge_attr.shape[0]
    IN, HID = W1.shape
    src = edge_index[0]
    dst = edge_index[1]
    K = 5
    Ec = E // K
    assert E % K == 0
    sc_gather = _make_sc_gather(N, Ec, D, x.dtype.name)
    g2 = ln_gamma.reshape(1, IN)
    bt = ln_beta.reshape(1, IN)
    w1b = W1.astype(jnp.bfloat16)
    b1r = b1.reshape(1, HID)
    w2b = W2.astype(jnp.bfloat16)
    b2r = b2.reshape(1, D)
    out = None
    for k in range(K):
        s_k = lax.slice_in_dim(src, k * Ec, (k + 1) * Ec)
        d_k = lax.slice_in_dim(dst, k * Ec, (k + 1) * Ec)
        hs_k, hd_k = sc_gather(x, s_k, d_k)
        tc = _make_tc_chunk(E, Ec, D, IN, HID, k, k > 0, x.dtype.name)
        args = (hs_k, hd_k, edge_attr, g2, bt, w1b, b1r, w2b, b2r)
        out = tc(*args) if k == 0 else tc(out, *args)
    return out
